# Initial kernel scaffold; baseline (speedup 1.0000x reference)
import jax
import jax.numpy as jnp
from jax import lax
from jax.experimental import pallas as pl
from jax.experimental.pallas import tpu as pltpu
from jax.experimental.pallas import tpu_sc as plsc

N = 50000
E = 800000
H = 64
HH = 32
G = 64
VPAD = 32
CH = 128
ACC_N = 50176
RPS = ACC_N // 16
WB = RPS // 8
E_PAD = 802816
CH_DEG = 196
CH_CONV = 392
BN = 2000
GRID = N // BN

_mesh = plsc.VectorSubcoreMesh(core_axis_name="c", subcore_axis_name="s")


def _deg_body(dst_hbm, zeros1, ones1, p0, p1, acc, idx_v, ones_v, stage):
    c = lax.axis_index("c")
    s = lax.axis_index("s")

    def run(out_ref):
        pltpu.sync_copy(zeros1, stage)
        pltpu.sync_copy(stage, acc.at[pl.ds(s * RPS, RPS)])
        pltpu.sync_copy(ones1, ones_v)
        plsc.subcore_barrier()
        base = (c * 16 + s) * (CH_DEG * CH)

        def chunk(k, carry):
            off = base + k * CH
            pltpu.sync_copy(dst_hbm.at[pl.ds(off, CH)], idx_v)
            pltpu.sync_copy(ones_v, acc.at[idx_v], add=True)
            return carry

        lax.fori_loop(0, CH_DEG, chunk, 0)
        plsc.subcore_barrier()
        pltpu.sync_copy(acc.at[pl.ds(s * RPS, RPS)], stage)
        pltpu.sync_copy(stage, out_ref.at[pl.ds(s * RPS, RPS)])

    @pl.when(c == 0)
    def _():
        run(p0)

    @pl.when(c == 1)
    def _():
        run(p1)


_deg_call = pl.kernel(
    _deg_body,
    out_type=(
        jax.ShapeDtypeStruct((ACC_N,), jnp.float32),
        jax.ShapeDtypeStruct((ACC_N,), jnp.float32),
    ),
    mesh=_mesh,
    scratch_types=[
        pltpu.VMEM_SHARED((ACC_N,), jnp.float32),
        pltpu.VMEM((CH,), jnp.int32),
        pltpu.VMEM((CH,), jnp.float32),
        pltpu.VMEM((RPS,), jnp.float32),
    ],
)


def _conv_body(uA, uB, src_hbm, dst_hbm, zrows, outA, outB,
               acc, sidx, didx, rows, stage, sem):
    c = lax.axis_index("c")
    s = lax.axis_index("s")

    def run(u_ref, out_ref):
        pltpu.sync_copy(zrows, stage)
        for j in range(8):
            pltpu.sync_copy(stage, acc.at[pl.ds(s * RPS + j * WB, WB)])
        plsc.subcore_barrier()
        base = s * (CH_CONV * CH)

        def chunk(k, carry):
            off = base + k * CH
            pltpu.sync_copy(src_hbm.at[pl.ds(off, CH)], sidx)
            pltpu.sync_copy(dst_hbm.at[pl.ds(off, CH)], didx)
            pltpu.async_copy(u_ref.at[sidx], rows, sem).wait()
            pltpu.sync_copy(rows, acc.at[didx], add=True)
            return carry

        lax.fori_loop(0, CH_CONV, chunk, 0)
        plsc.subcore_barrier()
        for j in range(8):
            off = s * RPS + j * WB
            pltpu.sync_copy(acc.at[pl.ds(off, WB)], stage)
            pltpu.sync_copy(stage, out_ref.at[pl.ds(off, WB)])

    @pl.when(c == 0)
    def _():
        run(uA, outA)

    @pl.when(c == 1)
    def _():
        run(uB, outB)


_conv_call = pl.kernel(
    _conv_body,
    out_type=(
        jax.ShapeDtypeStruct((ACC_N, HH), jnp.float32),
        jax.ShapeDtypeStruct((ACC_N, HH), jnp.float32),
    ),
    mesh=_mesh,
    scratch_types=[
        pltpu.VMEM_SHARED((ACC_N, HH), jnp.float32),
        pltpu.VMEM((CH,), jnp.int32),
        pltpu.VMEM((CH,), jnp.int32),
        pltpu.VMEM((CH, HH), jnp.float32),
        pltpu.VMEM((WB, HH), jnp.float32),
        pltpu.SemaphoreType.DMA,
    ],
)


def _prep_body(ids, p0, p1, embp, W1, uA, uB, dinv):
    deg = 1.0 + p0[...] + p1[...]
    di = lax.rsqrt(deg)
    oh = (ids[...] == lax.broadcasted_iota(jnp.int32, (BN, VPAD), 1))
    h0 = jnp.dot(oh.astype(jnp.float32), embp[...],
                 preferred_element_type=jnp.float32)
    t1 = jnp.dot(h0, W1[...], preferred_element_type=jnp.float32)
    u = t1 * di
    uA[...] = u[:, :HH]
    uB[...] = u[:, HH:]
    dinv[...] = di


_prep_call = pl.pallas_call(
    _prep_body,
    grid=(GRID,),
    in_specs=[
        pl.BlockSpec((BN, 1), lambda i: (i, 0)),
        pl.BlockSpec((BN, 1), lambda i: (i, 0)),
        pl.BlockSpec((BN, 1), lambda i: (i, 0)),
        pl.BlockSpec((VPAD, H), lambda i: (0, 0)),
        pl.BlockSpec((H, H), lambda i: (0, 0)),
    ],
    out_specs=[
        pl.BlockSpec((BN, HH), lambda i: (i, 0)),
        pl.BlockSpec((BN, HH), lambda i: (i, 0)),
        pl.BlockSpec((BN, 1), lambda i: (i, 0)),
    ],
    out_shape=[
        jax.ShapeDtypeStruct((N, HH), jnp.float32),
        jax.ShapeDtypeStruct((N, HH), jnp.float32),
        jax.ShapeDtypeStruct((N, 1), jnp.float32),
    ],
)


def _mid_body(aA, aB, uA, uB, dinv, b1, W2, oA, oB):
    di = dinv[...]
    acc = jnp.concatenate([aA[...], aB[...]], axis=1)
    u = jnp.concatenate([uA[...], uB[...]], axis=1)
    h1 = jnp.maximum(di * (acc + u) + b1[...], 0.0)
    hw = jnp.dot(h1, W2[...], preferred_element_type=jnp.float32)
    u2 = hw * di
    oA[...] = u2[:, :HH]
    oB[...] = u2[:, HH:]


_mid_call = pl.pallas_call(
    _mid_body,
    grid=(GRID,),
    in_specs=[
        pl.BlockSpec((BN, HH), lambda i: (i, 0)),
        pl.BlockSpec((BN, HH), lambda i: (i, 0)),
        pl.BlockSpec((BN, HH), lambda i: (i, 0)),
        pl.BlockSpec((BN, HH), lambda i: (i, 0)),
        pl.BlockSpec((BN, 1), lambda i: (i, 0)),
        pl.BlockSpec((1, H), lambda i: (0, 0)),
        pl.BlockSpec((H, H), lambda i: (0, 0)),
    ],
    out_specs=[
        pl.BlockSpec((BN, HH), lambda i: (i, 0)),
        pl.BlockSpec((BN, HH), lambda i: (i, 0)),
    ],
    out_shape=[
        jax.ShapeDtypeStruct((N, HH), jnp.float32),
        jax.ShapeDtypeStruct((N, HH), jnp.float32),
    ],
)


def _tail_body(aA, aB, uA, uB, dinv, b2, batch, W3, b3, w4r, b4,
               out, sums, cnts):
    i = pl.program_id(0)
    di = dinv[...]
    acc = jnp.concatenate([aA[...], aB[...]], axis=1)
    u = jnp.concatenate([uA[...], uB[...]], axis=1)
    h2 = jnp.maximum(di * (acc + u) + b2[...], 0.0)
    oh = (batch[...] == lax.broadcasted_iota(jnp.int32, (BN, G), 1))
    ohf = oh.astype(jnp.float32)
    ps = lax.dot_general(ohf, h2, (((0,), (0,)), ((), ())),
                         preferred_element_type=jnp.float32)
    pc = lax.dot_general(ohf, jnp.ones((BN, 1), jnp.float32),
                         (((0,), (0,)), ((), ())),
                         preferred_element_type=jnp.float32)

    @pl.when(i == 0)
    def _():
        sums[...] = ps
        cnts[...] = pc

    @pl.when(i > 0)
    def _():
        sums[...] += ps
        cnts[...] += pc

    @pl.when(i == GRID - 1)
    def _():
        pooled = sums[...] / jnp.maximum(cnts[...], 1.0)
        hid = jnp.maximum(
            jnp.dot(pooled, W3[...], preferred_element_type=jnp.float32)
            + b3[...], 0.0)
        out[...] = jnp.sum(hid * w4r[...], axis=1, keepdims=True) + b4[...]


_tail_call = pl.pallas_call(
    _tail_body,
    grid=(GRID,),
    in_specs=[
        pl.BlockSpec((BN, HH), lambda i: (i, 0)),
        pl.BlockSpec((BN, HH), lambda i: (i, 0)),
        pl.BlockSpec((BN, HH), lambda i: (i, 0)),
        pl.BlockSpec((BN, HH), lambda i: (i, 0)),
        pl.BlockSpec((BN, 1), lambda i: (i, 0)),
        pl.BlockSpec((1, H), lambda i: (0, 0)),
        pl.BlockSpec((BN, 1), lambda i: (i, 0)),
        pl.BlockSpec((H, H), lambda i: (0, 0)),
        pl.BlockSpec((1, H), lambda i: (0, 0)),
        pl.BlockSpec((1, H), lambda i: (0, 0)),
        pl.BlockSpec((1, 1), lambda i: (0, 0)),
    ],
    out_specs=[pl.BlockSpec((G, 1), lambda i: (0, 0))],
    out_shape=[jax.ShapeDtypeStruct((G, 1), jnp.float32)],
    scratch_shapes=[
        pltpu.VMEM((G, G), jnp.float32),
        pltpu.VMEM((G, 1), jnp.float32),
    ],
)


def kernel(x, edge_index, batch, emb, W1, b1, W2, b2, W3, b3, W4, b4):
    src = edge_index[0]
    dst = edge_index[1]
    srcp = jnp.concatenate([src, jnp.zeros((E_PAD - E,), jnp.int32)])
    dstp = jnp.concatenate([dst, jnp.full((E_PAD - E,), N, jnp.int32)])

    zeros1 = jnp.zeros((RPS,), jnp.float32)
    ones1 = jnp.ones((CH,), jnp.float32)
    zrows = jnp.zeros((WB, HH), jnp.float32)

    p0, p1 = _deg_call(dstp, zeros1, ones1)
    p0 = p0[:N].reshape(N, 1)
    p1 = p1[:N].reshape(N, 1)

    embp = jnp.zeros((VPAD, H), jnp.float32).at[:emb.shape[0]].set(emb)
    uA, uB, dinv = _prep_call(x, p0, p1, embp, W1)

    a1A, a1B = _conv_call(uA, uB, srcp, dstp, zrows)
    u2A, u2B = _mid_call(a1A[:N], a1B[:N], uA, uB, dinv,
                         b1.reshape(1, H), W2)

    a2A, a2B = _conv_call(u2A, u2B, srcp, dstp, zrows)
    (out,) = _tail_call(a2A[:N], a2B[:N], u2A, u2B, dinv,
                        b2.reshape(1, H), batch.reshape(N, 1),
                        W3, b3.reshape(1, H), W4.reshape(1, H),
                        b4.reshape(1, 1))
    return out


# speedup vs baseline: 11.7686x; 11.7686x over previous
import jax
import jax.numpy as jnp
from jax import lax
from jax.experimental import pallas as pl
from jax.experimental.pallas import tpu as pltpu
from jax.experimental.pallas import tpu_sc as plsc

N = 50000
E = 800000
H = 64
HH = 32
G = 64
VPAD = 32
CH = 128
ACC_N = 50176
RPS = ACC_N // 16
WB = RPS // 8
E_PAD = 802816
CH_DEG = 196
CH_CONV = 392
BN = 2000
GRID = N // BN

_mesh = plsc.VectorSubcoreMesh(core_axis_name="c", subcore_axis_name="s")
_sc_params = pltpu.CompilerParams(use_tc_tiling_on_sc=False)


def _deg_body(dst_hbm, zeros1, ones1, p0, p1, acc, idx_v, ones_v, stage):
    c = lax.axis_index("c")
    s = lax.axis_index("s")

    def run(out_ref):
        pltpu.sync_copy(zeros1, stage)
        pltpu.sync_copy(stage, acc.at[pl.ds(s * RPS, RPS)])
        pltpu.sync_copy(ones1, ones_v)
        plsc.subcore_barrier()
        base = (c * 16 + s) * (CH_DEG * CH)

        def chunk(k, carry):
            off = base + k * CH
            pltpu.sync_copy(dst_hbm.at[pl.ds(off, CH)], idx_v)
            pltpu.sync_copy(ones_v, acc.at[idx_v], add=True)
            return carry

        lax.fori_loop(0, CH_DEG, chunk, 0)
        plsc.subcore_barrier()
        pltpu.sync_copy(acc.at[pl.ds(s * RPS, RPS)], stage)
        pltpu.sync_copy(stage, out_ref.at[pl.ds(s * RPS, RPS)])

    @pl.when(c == 0)
    def _():
        run(p0)

    @pl.when(c == 1)
    def _():
        run(p1)


_deg_call = pl.kernel(
    _deg_body,
    out_type=(
        jax.ShapeDtypeStruct((ACC_N,), jnp.float32),
        jax.ShapeDtypeStruct((ACC_N,), jnp.float32),
    ),
    mesh=_mesh,
    scratch_types=[
        pltpu.VMEM_SHARED((ACC_N,), jnp.float32),
        pltpu.VMEM((CH,), jnp.int32),
        pltpu.VMEM((CH,), jnp.float32),
        pltpu.VMEM((RPS,), jnp.float32),
    ],
    compiler_params=_sc_params,
)


def _conv_body(uA, uB, src_hbm, dst_hbm, zrows, outA, outB,
               acc, sidx, didx, rows, stage, sem):
    c = lax.axis_index("c")
    s = lax.axis_index("s")

    def run(u_ref, out_ref):
        pltpu.sync_copy(zrows, stage)
        for j in range(8):
            pltpu.sync_copy(stage, acc.at[pl.ds(s * RPS + j * WB, WB)])
        plsc.subcore_barrier()
        base = s * (CH_CONV * CH)

        def chunk(k, carry):
            off = base + k * CH
            pltpu.sync_copy(src_hbm.at[pl.ds(off, CH)], sidx)
            pltpu.sync_copy(dst_hbm.at[pl.ds(off, CH)], didx)
            pltpu.async_copy(u_ref.at[sidx], rows, sem).wait()
            pltpu.sync_copy(rows, acc.at[didx], add=True)
            return carry

        lax.fori_loop(0, CH_CONV, chunk, 0)
        plsc.subcore_barrier()
        for j in range(8):
            off = s * RPS + j * WB
            pltpu.sync_copy(acc.at[pl.ds(off, WB)], stage)
            pltpu.sync_copy(stage, out_ref.at[pl.ds(off, WB)])

    @pl.when(c == 0)
    def _():
        run(uA, outA)

    @pl.when(c == 1)
    def _():
        run(uB, outB)


_conv_call = pl.kernel(
    _conv_body,
    out_type=(
        jax.ShapeDtypeStruct((ACC_N, HH), jnp.float32),
        jax.ShapeDtypeStruct((ACC_N, HH), jnp.float32),
    ),
    mesh=_mesh,
    scratch_types=[
        pltpu.VMEM_SHARED((ACC_N, HH), jnp.float32),
        pltpu.VMEM((CH,), jnp.int32),
        pltpu.VMEM((CH,), jnp.int32),
        pltpu.VMEM((CH, HH), jnp.float32),
        pltpu.VMEM((WB, HH), jnp.float32),
        pltpu.SemaphoreType.DMA,
    ],
    compiler_params=_sc_params,
)


def _prep_body(ids, p0, p1, embp, W1, uA, uB, dinv):
    deg = 1.0 + p0[...] + p1[...]
    di = lax.rsqrt(deg)
    oh = (ids[...] == lax.broadcasted_iota(jnp.int32, (BN, VPAD), 1))
    h0 = jnp.dot(oh.astype(jnp.float32), embp[...],
                 preferred_element_type=jnp.float32)
    t1 = jnp.dot(h0, W1[...], preferred_element_type=jnp.float32)
    u = t1 * di
    uA[...] = u[:, :HH]
    uB[...] = u[:, HH:]
    dinv[...] = di


_prep_call = pl.pallas_call(
    _prep_body,
    grid=(GRID,),
    in_specs=[
        pl.BlockSpec((BN, 1), lambda i: (i, 0)),
        pl.BlockSpec((BN, 1), lambda i: (i, 0)),
        pl.BlockSpec((BN, 1), lambda i: (i, 0)),
        pl.BlockSpec((VPAD, H), lambda i: (0, 0)),
        pl.BlockSpec((H, H), lambda i: (0, 0)),
    ],
    out_specs=[
        pl.BlockSpec((BN, HH), lambda i: (i, 0)),
        pl.BlockSpec((BN, HH), lambda i: (i, 0)),
        pl.BlockSpec((BN, 1), lambda i: (i, 0)),
    ],
    out_shape=[
        jax.ShapeDtypeStruct((N, HH), jnp.float32),
        jax.ShapeDtypeStruct((N, HH), jnp.float32),
        jax.ShapeDtypeStruct((N, 1), jnp.float32),
    ],
)


def _mid_body(aA, aB, uA, uB, dinv, b1, W2, oA, oB):
    di = dinv[...]
    acc = jnp.concatenate([aA[...], aB[...]], axis=1)
    u = jnp.concatenate([uA[...], uB[...]], axis=1)
    h1 = jnp.maximum(di * (acc + u) + b1[...], 0.0)
    hw = jnp.dot(h1, W2[...], preferred_element_type=jnp.float32)
    u2 = hw * di
    oA[...] = u2[:, :HH]
    oB[...] = u2[:, HH:]


_mid_call = pl.pallas_call(
    _mid_body,
    grid=(GRID,),
    in_specs=[
        pl.BlockSpec((BN, HH), lambda i: (i, 0)),
        pl.BlockSpec((BN, HH), lambda i: (i, 0)),
        pl.BlockSpec((BN, HH), lambda i: (i, 0)),
        pl.BlockSpec((BN, HH), lambda i: (i, 0)),
        pl.BlockSpec((BN, 1), lambda i: (i, 0)),
        pl.BlockSpec((1, H), lambda i: (0, 0)),
        pl.BlockSpec((H, H), lambda i: (0, 0)),
    ],
    out_specs=[
        pl.BlockSpec((BN, HH), lambda i: (i, 0)),
        pl.BlockSpec((BN, HH), lambda i: (i, 0)),
    ],
    out_shape=[
        jax.ShapeDtypeStruct((N, HH), jnp.float32),
        jax.ShapeDtypeStruct((N, HH), jnp.float32),
    ],
)


def _tail_body(aA, aB, uA, uB, dinv, b2, batch, W3, b3, w4r, b4,
               out, sums, cnts):
    i = pl.program_id(0)
    di = dinv[...]
    acc = jnp.concatenate([aA[...], aB[...]], axis=1)
    u = jnp.concatenate([uA[...], uB[...]], axis=1)
    h2 = jnp.maximum(di * (acc + u) + b2[...], 0.0)
    oh = (batch[...] == lax.broadcasted_iota(jnp.int32, (BN, G), 1))
    ohf = oh.astype(jnp.float32)
    ps = lax.dot_general(ohf, h2, (((0,), (0,)), ((), ())),
                         preferred_element_type=jnp.float32)
    pc = lax.dot_general(ohf, jnp.ones((BN, 1), jnp.float32),
                         (((0,), (0,)), ((), ())),
                         preferred_element_type=jnp.float32)

    @pl.when(i == 0)
    def _():
        sums[...] = ps
        cnts[...] = pc

    @pl.when(i > 0)
    def _():
        sums[...] += ps
        cnts[...] += pc

    @pl.when(i == GRID - 1)
    def _():
        pooled = sums[...] / jnp.maximum(cnts[...], 1.0)
        hid = jnp.maximum(
            jnp.dot(pooled, W3[...], preferred_element_type=jnp.float32)
            + b3[...], 0.0)
        out[...] = jnp.sum(hid * w4r[...], axis=1, keepdims=True) + b4[...]


_tail_call = pl.pallas_call(
    _tail_body,
    grid=(GRID,),
    in_specs=[
        pl.BlockSpec((BN, HH), lambda i: (i, 0)),
        pl.BlockSpec((BN, HH), lambda i: (i, 0)),
        pl.BlockSpec((BN, HH), lambda i: (i, 0)),
        pl.BlockSpec((BN, HH), lambda i: (i, 0)),
        pl.BlockSpec((BN, 1), lambda i: (i, 0)),
        pl.BlockSpec((1, H), lambda i: (0, 0)),
        pl.BlockSpec((BN, 1), lambda i: (i, 0)),
        pl.BlockSpec((H, H), lambda i: (0, 0)),
        pl.BlockSpec((1, H), lambda i: (0, 0)),
        pl.BlockSpec((1, H), lambda i: (0, 0)),
        pl.BlockSpec((1, 1), lambda i: (0, 0)),
    ],
    out_specs=[pl.BlockSpec((G, 1), lambda i: (0, 0))],
    out_shape=[jax.ShapeDtypeStruct((G, 1), jnp.float32)],
    scratch_shapes=[
        pltpu.VMEM((G, G), jnp.float32),
        pltpu.VMEM((G, 1), jnp.float32),
    ],
)


def kernel(x, edge_index, batch, emb, W1, b1, W2, b2, W3, b3, W4, b4):
    src = edge_index[0]
    dst = edge_index[1]
    srcp = jnp.concatenate([src, jnp.zeros((E_PAD - E,), jnp.int32)])
    dstp = jnp.concatenate([dst, jnp.full((E_PAD - E,), N, jnp.int32)])

    zeros1 = jnp.zeros((RPS,), jnp.float32)
    ones1 = jnp.ones((CH,), jnp.float32)
    zrows = jnp.zeros((WB, HH), jnp.float32)

    p0, p1 = _deg_call(dstp, zeros1, ones1)
    p0 = p0[:N].reshape(N, 1)
    p1 = p1[:N].reshape(N, 1)

    embp = jnp.zeros((VPAD, H), jnp.float32).at[:emb.shape[0]].set(emb)
    uA, uB, dinv = _prep_call(x, p0, p1, embp, W1)

    a1A, a1B = _conv_call(uA, uB, srcp, dstp, zrows)
    u2A, u2B = _mid_call(a1A[:N], a1B[:N], uA, uB, dinv,
                         b1.reshape(1, H), W2)

    a2A, a2B = _conv_call(u2A, u2B, srcp, dstp, zrows)
    (out,) = _tail_call(a2A[:N], a2B[:N], u2A, u2B, dinv,
                        b2.reshape(1, H), batch.reshape(N, 1),
                        W3, b3.reshape(1, H), W4.reshape(1, H),
                        b4.reshape(1, 1))
    return out


# speedup vs baseline: 20.9651x; 1.7814x over previous
import jax
import jax.numpy as jnp
from jax import lax
from jax.experimental import pallas as pl
from jax.experimental.pallas import tpu as pltpu
from jax.experimental.pallas import tpu_sc as plsc

N = 50000
E = 800000
H = 64
HH = 32
G = 64
VPAD = 32
CH = 128
ACC_N = 50176
RPS = ACC_N // 16
WB = RPS // 8
E_PAD = 802816
CH_DEG = 196
CH_CONV = 392
BN = 2000
GRID = N // BN

_mesh = plsc.VectorSubcoreMesh(core_axis_name="c", subcore_axis_name="s")
_sc_params = pltpu.CompilerParams(use_tc_tiling_on_sc=False)


def _deg_body(dst_hbm, zeros1, ones1, p0, p1, acc, idx_v, ones_v, stage):
    c = lax.axis_index("c")
    s = lax.axis_index("s")

    def run(out_ref):
        pltpu.sync_copy(zeros1, stage)
        pltpu.sync_copy(stage, acc.at[pl.ds(s * RPS, RPS)])
        pltpu.sync_copy(ones1, ones_v)
        plsc.subcore_barrier()
        rbase = (c * 16 + s) * CH_DEG

        def chunk(k, carry):
            r0 = rbase + k * 4
            pltpu.sync_copy(dst_hbm.at[pl.ds(r0, 4)], idx_v)
            for j in range(4):
                pltpu.sync_copy(ones_v, acc.at[idx_v.at[j]], add=True)
            return carry

        lax.fori_loop(0, CH_DEG // 4, chunk, 0)
        plsc.subcore_barrier()
        pltpu.sync_copy(acc.at[pl.ds(s * RPS, RPS)], stage)
        pltpu.sync_copy(stage, out_ref.at[pl.ds(s * RPS, RPS)])

    @pl.when(c == 0)
    def _():
        run(p0)

    @pl.when(c == 1)
    def _():
        run(p1)


_deg_call = pl.kernel(
    _deg_body,
    out_type=(
        jax.ShapeDtypeStruct((ACC_N,), jnp.float32),
        jax.ShapeDtypeStruct((ACC_N,), jnp.float32),
    ),
    mesh=_mesh,
    scratch_types=[
        pltpu.VMEM_SHARED((ACC_N,), jnp.float32),
        pltpu.VMEM((4, CH), jnp.int32),
        pltpu.VMEM((CH,), jnp.float32),
        pltpu.VMEM((RPS,), jnp.float32),
    ],
    compiler_params=_sc_params,
)


GRP = 2
NG = CH_CONV // GRP
GROWS = GRP * CH


def _conv_body(uA, uB, src_hbm, dst_hbm, zrows, outA, outB,
               acc, sidx0, sidx1, didx0, didx1, rows0, rows1,
               sem0, sem1):
    c = lax.axis_index("c")
    s = lax.axis_index("s")

    def run(u_ref, out_ref):
        for j in range(8):
            pltpu.sync_copy(zrows, acc.at[pl.ds(s * RPS + j * WB, WB)])
        plsc.subcore_barrier()
        rbase = s * CH_CONV

        def load_grp(sidx, didx, rows, sem, g):
            r0 = rbase + g * GRP
            pltpu.sync_copy(src_hbm.at[pl.ds(r0, GRP)], sidx)
            pltpu.sync_copy(dst_hbm.at[pl.ds(r0, GRP)], didx)
            for j in range(GRP):
                pltpu.async_copy(u_ref.at[sidx.at[j]],
                                 rows.at[pl.ds(j * CH, CH)], sem)

        def drain_scatter(didx, rows, sem):
            pltpu.make_async_copy(u_ref.at[pl.ds(0, GROWS)], rows, sem).wait()
            for j in range(GRP):
                pltpu.sync_copy(rows.at[pl.ds(j * CH, CH)],
                                acc.at[didx.at[j]], add=True)

        load_grp(sidx0, didx0, rows0, sem0, 0)

        def body(t, carry):
            g0 = 2 * t
            load_grp(sidx1, didx1, rows1, sem1, g0 + 1)
            drain_scatter(didx0, rows0, sem0)

            @pl.when(g0 + 2 < NG)
            def _():
                load_grp(sidx0, didx0, rows0, sem0, g0 + 2)

            drain_scatter(didx1, rows1, sem1)
            return carry

        lax.fori_loop(0, NG // 2, body, 0)
        plsc.subcore_barrier()
        pltpu.sync_copy(acc.at[pl.ds(s * RPS, RPS)],
                        out_ref.at[pl.ds(s * RPS, RPS)])

    @pl.when(c == 0)
    def _():
        run(uA, outA)

    @pl.when(c == 1)
    def _():
        run(uB, outB)


_conv_call = pl.kernel(
    _conv_body,
    out_type=(
        jax.ShapeDtypeStruct((ACC_N, HH), jnp.float32),
        jax.ShapeDtypeStruct((ACC_N, HH), jnp.float32),
    ),
    mesh=_mesh,
    scratch_types=[
        pltpu.VMEM_SHARED((ACC_N, HH), jnp.float32),
        pltpu.VMEM((GRP, CH), jnp.int32),
        pltpu.VMEM((GRP, CH), jnp.int32),
        pltpu.VMEM((GRP, CH), jnp.int32),
        pltpu.VMEM((GRP, CH), jnp.int32),
        pltpu.VMEM((GROWS, HH), jnp.float32),
        pltpu.VMEM((GROWS, HH), jnp.float32),
        pltpu.SemaphoreType.DMA,
        pltpu.SemaphoreType.DMA,
    ],
    compiler_params=_sc_params,
)


def _prep_body(ids, p0, p1, embp, W1, uA, uB, dinv):
    deg = 1.0 + p0[...] + p1[...]
    di = lax.rsqrt(deg)
    oh = (ids[...] == lax.broadcasted_iota(jnp.int32, (BN, VPAD), 1))
    h0 = jnp.dot(oh.astype(jnp.float32), embp[...],
                 preferred_element_type=jnp.float32)
    t1 = jnp.dot(h0, W1[...], preferred_element_type=jnp.float32)
    u = t1 * di
    uA[...] = u[:, :HH]
    uB[...] = u[:, HH:]
    dinv[...] = di


_prep_call = pl.pallas_call(
    _prep_body,
    grid=(GRID,),
    in_specs=[
        pl.BlockSpec((BN, 1), lambda i: (i, 0)),
        pl.BlockSpec((BN, 1), lambda i: (i, 0)),
        pl.BlockSpec((BN, 1), lambda i: (i, 0)),
        pl.BlockSpec((VPAD, H), lambda i: (0, 0)),
        pl.BlockSpec((H, H), lambda i: (0, 0)),
    ],
    out_specs=[
        pl.BlockSpec((BN, HH), lambda i: (i, 0)),
        pl.BlockSpec((BN, HH), lambda i: (i, 0)),
        pl.BlockSpec((BN, 1), lambda i: (i, 0)),
    ],
    out_shape=[
        jax.ShapeDtypeStruct((N, HH), jnp.float32),
        jax.ShapeDtypeStruct((N, HH), jnp.float32),
        jax.ShapeDtypeStruct((N, 1), jnp.float32),
    ],
)


def _mid_body(aA, aB, uA, uB, dinv, b1, W2, oA, oB):
    di = dinv[...]
    acc = jnp.concatenate([aA[...], aB[...]], axis=1)
    u = jnp.concatenate([uA[...], uB[...]], axis=1)
    h1 = jnp.maximum(di * (acc + u) + b1[...], 0.0)
    hw = jnp.dot(h1, W2[...], preferred_element_type=jnp.float32)
    u2 = hw * di
    oA[...] = u2[:, :HH]
    oB[...] = u2[:, HH:]


_mid_call = pl.pallas_call(
    _mid_body,
    grid=(GRID,),
    in_specs=[
        pl.BlockSpec((BN, HH), lambda i: (i, 0)),
        pl.BlockSpec((BN, HH), lambda i: (i, 0)),
        pl.BlockSpec((BN, HH), lambda i: (i, 0)),
        pl.BlockSpec((BN, HH), lambda i: (i, 0)),
        pl.BlockSpec((BN, 1), lambda i: (i, 0)),
        pl.BlockSpec((1, H), lambda i: (0, 0)),
        pl.BlockSpec((H, H), lambda i: (0, 0)),
    ],
    out_specs=[
        pl.BlockSpec((BN, HH), lambda i: (i, 0)),
        pl.BlockSpec((BN, HH), lambda i: (i, 0)),
    ],
    out_shape=[
        jax.ShapeDtypeStruct((N, HH), jnp.float32),
        jax.ShapeDtypeStruct((N, HH), jnp.float32),
    ],
)


def _tail_body(aA, aB, uA, uB, dinv, b2, batch, W3, b3, w4r, b4,
               out, sums, cnts):
    i = pl.program_id(0)
    di = dinv[...]
    acc = jnp.concatenate([aA[...], aB[...]], axis=1)
    u = jnp.concatenate([uA[...], uB[...]], axis=1)
    h2 = jnp.maximum(di * (acc + u) + b2[...], 0.0)
    oh = (batch[...] == lax.broadcasted_iota(jnp.int32, (BN, G), 1))
    ohf = oh.astype(jnp.float32)
    ps = lax.dot_general(ohf, h2, (((0,), (0,)), ((), ())),
                         preferred_element_type=jnp.float32)
    pc = lax.dot_general(ohf, jnp.ones((BN, 1), jnp.float32),
                         (((0,), (0,)), ((), ())),
                         preferred_element_type=jnp.float32)

    @pl.when(i == 0)
    def _():
        sums[...] = ps
        cnts[...] = pc

    @pl.when(i > 0)
    def _():
        sums[...] += ps
        cnts[...] += pc

    @pl.when(i == GRID - 1)
    def _():
        pooled = sums[...] / jnp.maximum(cnts[...], 1.0)
        hid = jnp.maximum(
            jnp.dot(pooled, W3[...], preferred_element_type=jnp.float32)
            + b3[...], 0.0)
        out[...] = jnp.sum(hid * w4r[...], axis=1, keepdims=True) + b4[...]


_tail_call = pl.pallas_call(
    _tail_body,
    grid=(GRID,),
    in_specs=[
        pl.BlockSpec((BN, HH), lambda i: (i, 0)),
        pl.BlockSpec((BN, HH), lambda i: (i, 0)),
        pl.BlockSpec((BN, HH), lambda i: (i, 0)),
        pl.BlockSpec((BN, HH), lambda i: (i, 0)),
        pl.BlockSpec((BN, 1), lambda i: (i, 0)),
        pl.BlockSpec((1, H), lambda i: (0, 0)),
        pl.BlockSpec((BN, 1), lambda i: (i, 0)),
        pl.BlockSpec((H, H), lambda i: (0, 0)),
        pl.BlockSpec((1, H), lambda i: (0, 0)),
        pl.BlockSpec((1, H), lambda i: (0, 0)),
        pl.BlockSpec((1, 1), lambda i: (0, 0)),
    ],
    out_specs=[pl.BlockSpec((G, 1), lambda i: (0, 0))],
    out_shape=[jax.ShapeDtypeStruct((G, 1), jnp.float32)],
    scratch_shapes=[
        pltpu.VMEM((G, G), jnp.float32),
        pltpu.VMEM((G, 1), jnp.float32),
    ],
)


def kernel(x, edge_index, batch, emb, W1, b1, W2, b2, W3, b3, W4, b4):
    src = edge_index[0]
    dst = edge_index[1]
    srcp = jnp.concatenate(
        [src, jnp.zeros((E_PAD - E,), jnp.int32)]).reshape(E_PAD // CH, CH)
    dstp = jnp.concatenate(
        [dst, jnp.full((E_PAD - E,), N, jnp.int32)]).reshape(E_PAD // CH, CH)

    zeros1 = jnp.zeros((RPS,), jnp.float32)
    ones1 = jnp.ones((CH,), jnp.float32)
    zrows = jnp.zeros((WB, HH), jnp.float32)

    p0, p1 = _deg_call(dstp, zeros1, ones1)
    p0 = p0[:N].reshape(N, 1)
    p1 = p1[:N].reshape(N, 1)

    embp = jnp.zeros((VPAD, H), jnp.float32).at[:emb.shape[0]].set(emb)
    uA, uB, dinv = _prep_call(x, p0, p1, embp, W1)

    a1A, a1B = _conv_call(uA, uB, srcp, dstp, zrows)
    u2A, u2B = _mid_call(a1A[:N], a1B[:N], uA, uB, dinv,
                         b1.reshape(1, H), W2)

    a2A, a2B = _conv_call(u2A, u2B, srcp, dstp, zrows)
    (out,) = _tail_call(a2A[:N], a2B[:N], u2A, u2B, dinv,
                        b2.reshape(1, H), batch.reshape(N, 1),
                        W3, b3.reshape(1, H), W4.reshape(1, H),
                        b4.reshape(1, 1))
    return out


# speedup vs baseline: 26.0303x; 1.2416x over previous
import jax
import jax.numpy as jnp
from jax import lax
from jax.experimental import pallas as pl
from jax.experimental.pallas import tpu as pltpu
from jax.experimental.pallas import tpu_sc as plsc

N = 50000
E = 800000
H = 64
HH = 32
G = 64
VPAD = 32
CH = 128
ACC_N = 50176
RPS = ACC_N // 16
WB = RPS // 8
E_PAD = 802816
CH_DEG = 196
CH_CONV = 392
BN = 2000
GRID = N // BN

_mesh = plsc.VectorSubcoreMesh(core_axis_name="c", subcore_axis_name="s")
_sc_params = pltpu.CompilerParams(use_tc_tiling_on_sc=False)
_sc_params_nl = pltpu.CompilerParams(use_tc_tiling_on_sc=False,
                                     needs_layout_passes=False)


def _deg_body(dst_hbm, zeros1, ones1, p0, p1, acc, idx_v, ones_v, stage):
    c = lax.axis_index("c")
    s = lax.axis_index("s")

    def run(out_ref):
        pltpu.sync_copy(zeros1, stage)
        pltpu.sync_copy(stage, acc.at[pl.ds(s * RPS, RPS)])
        pltpu.sync_copy(ones1, ones_v)
        plsc.subcore_barrier()
        rbase = (c * 16 + s) * CH_DEG

        def chunk(k, carry):
            r0 = rbase + k * 4
            pltpu.sync_copy(dst_hbm.at[pl.ds(r0, 4)], idx_v)
            for j in range(4):
                pltpu.sync_copy(ones_v, acc.at[idx_v.at[j]], add=True)
            return carry

        lax.fori_loop(0, CH_DEG // 4, chunk, 0)
        plsc.subcore_barrier()
        pltpu.sync_copy(acc.at[pl.ds(s * RPS, RPS)], stage)
        pltpu.sync_copy(stage, out_ref.at[pl.ds(s * RPS, RPS)])

    @pl.when(c == 0)
    def _():
        run(p0)

    @pl.when(c == 1)
    def _():
        run(p1)


_deg_call = pl.kernel(
    _deg_body,
    out_type=(
        jax.ShapeDtypeStruct((ACC_N,), jnp.float32),
        jax.ShapeDtypeStruct((ACC_N,), jnp.float32),
    ),
    mesh=_mesh,
    scratch_types=[
        pltpu.VMEM_SHARED((ACC_N,), jnp.float32),
        pltpu.VMEM((4, CH), jnp.int32),
        pltpu.VMEM((CH,), jnp.float32),
        pltpu.VMEM((RPS,), jnp.float32),
    ],
    compiler_params=_sc_params,
)


GRP = 2
NG = CH_CONV // GRP
GROWS = GRP * CH


def _conv_body(uA, uB, src_hbm, dst_hbm, zrows, outA, outB,
               acc, sidx0, sidx1, didx0, didx1, rows0, rows1,
               sem0, sem1):
    c = lax.axis_index("c")
    s = lax.axis_index("s")

    def run(u_ref, out_ref):
        for j in range(8):
            pltpu.sync_copy(zrows, acc.at[pl.ds(s * RPS + j * WB, WB)])
        plsc.subcore_barrier()
        rbase = s * CH_CONV

        def load_grp(sidx, didx, rows, sem, g):
            r0 = rbase + g * GRP
            pltpu.sync_copy(src_hbm.at[pl.ds(r0, GRP)], sidx)
            pltpu.sync_copy(dst_hbm.at[pl.ds(r0, GRP)], didx)
            for j in range(GRP):
                pltpu.async_copy(u_ref.at[sidx.at[j]],
                                 rows.at[pl.ds(j * CH, CH)], sem)

        def drain_scatter(didx, rows, sem):
            pltpu.make_async_copy(u_ref.at[pl.ds(0, GROWS)], rows, sem).wait()
            for j in range(GRP):
                pltpu.sync_copy(rows.at[pl.ds(j * CH, CH)],
                                acc.at[didx.at[j]], add=True)

        load_grp(sidx0, didx0, rows0, sem0, 0)

        def body(t, carry):
            g0 = 2 * t
            load_grp(sidx1, didx1, rows1, sem1, g0 + 1)
            drain_scatter(didx0, rows0, sem0)

            @pl.when(g0 + 2 < NG)
            def _():
                load_grp(sidx0, didx0, rows0, sem0, g0 + 2)

            drain_scatter(didx1, rows1, sem1)
            return carry

        lax.fori_loop(0, NG // 2, body, 0)
        plsc.subcore_barrier()
        pltpu.sync_copy(acc.at[pl.ds(s * RPS, RPS)],
                        out_ref.at[pl.ds(s * RPS, RPS)])

    @pl.when(c == 0)
    def _():
        run(uA, outA)

    @pl.when(c == 1)
    def _():
        run(uB, outB)


_conv_call = pl.kernel(
    _conv_body,
    out_type=(
        jax.ShapeDtypeStruct((ACC_N, HH), jnp.float32),
        jax.ShapeDtypeStruct((ACC_N, HH), jnp.float32),
    ),
    mesh=_mesh,
    scratch_types=[
        pltpu.VMEM_SHARED((ACC_N, HH), jnp.float32),
        pltpu.VMEM((GRP, CH), jnp.int32),
        pltpu.VMEM((GRP, CH), jnp.int32),
        pltpu.VMEM((GRP, CH), jnp.int32),
        pltpu.VMEM((GRP, CH), jnp.int32),
        pltpu.VMEM((GROWS, HH), jnp.float32),
        pltpu.VMEM((GROWS, HH), jnp.float32),
        pltpu.SemaphoreType.DMA,
        pltpu.SemaphoreType.DMA,
    ],
    compiler_params=_sc_params,
)


def _vrsqrt(d):
    i = plsc.bitcast(d, jnp.int32)
    i = jnp.int32(0x5F3759DF) - lax.shift_right_arithmetic(i, 1)
    y = plsc.bitcast(i, jnp.float32)
    hx = 0.5 * d
    for _ in range(3):
        y = y * (1.5 - hx * y * y)
    return y


def _conv1_body(src_hbm, dst_hbm, p0h, p1h, ids1d, zf, A0, A1, acc,
                sidx0, sidx1, didx0, didx1, pv00, pv01, pv10, pv11,
                colv0, colv1, valv0, valv1, fidx0, fidx1, sem0, sem1):
    c = lax.axis_index("c")
    s = lax.axis_index("s")
    bufs = ((sidx0, didx0, pv00, pv10, colv0, valv0, fidx0, sem0),
            (sidx1, didx1, pv01, pv11, colv1, valv1, fidx1, sem1))

    def run(out_ref):
        pltpu.sync_copy(zf, acc.at[pl.ds(s * RPS * HH, RPS * HH)])
        plsc.subcore_barrier()
        rbase = (c * 16 + s) * CH_DEG

        def issue(b, k):
            sidx, didx, pv0, pv1, colv, valv, fidx, sem = bufs[b]
            r = rbase + k
            pltpu.sync_copy(src_hbm.at[pl.ds(r, 1)], sidx)
            pltpu.sync_copy(dst_hbm.at[pl.ds(r, 1)], didx)
            pltpu.async_copy(p0h.at[sidx.at[0]], pv0, sem)
            pltpu.async_copy(p1h.at[sidx.at[0]], pv1, sem)
            pltpu.async_copy(ids1d.at[sidx.at[0]], colv, sem)

        def process(b, k):
            sidx, didx, pv0, pv1, colv, valv, fidx, sem = bufs[b]
            pltpu.make_async_copy(p0h.at[pl.ds(0, CH)], pv0, sem).wait()
            pltpu.make_async_copy(p1h.at[pl.ds(0, CH)], pv1, sem).wait()
            pltpu.make_async_copy(ids1d.at[pl.ds(0, CH)], colv, sem).wait()
            for q in range(8):
                sl = pl.ds(16 * q, 16)
                d = 1.0 + pv0[sl] + pv1[sl]
                valv[sl] = _vrsqrt(d)
                fidx[sl] = didx[0, sl] * HH + colv[sl]
            pltpu.sync_copy(valv, acc.at[fidx], add=True)

        issue(0, 0)

        def body(t, carry):
            k0 = 2 * t
            issue(1, k0 + 1)
            process(0, k0)

            @pl.when(k0 + 2 < CH_DEG)
            def _():
                issue(0, k0 + 2)

            process(1, k0 + 1)
            return carry

        lax.fori_loop(0, CH_DEG // 2, body, 0)
        plsc.subcore_barrier()
        pltpu.sync_copy(acc.at[pl.ds(s * RPS * HH, RPS * HH)],
                        out_ref.at[pl.ds(s * RPS * HH, RPS * HH)])

    @pl.when(c == 0)
    def _():
        run(A0)

    @pl.when(c == 1)
    def _():
        run(A1)


_conv1_call = pl.kernel(
    _conv1_body,
    out_type=(
        jax.ShapeDtypeStruct((ACC_N * HH,), jnp.float32),
        jax.ShapeDtypeStruct((ACC_N * HH,), jnp.float32),
    ),
    mesh=_mesh,
    scratch_types=[
        pltpu.VMEM_SHARED((ACC_N * HH,), jnp.float32),
        pltpu.VMEM((1, CH), jnp.int32),
        pltpu.VMEM((1, CH), jnp.int32),
        pltpu.VMEM((1, CH), jnp.int32),
        pltpu.VMEM((1, CH), jnp.int32),
        pltpu.VMEM((CH,), jnp.float32),
        pltpu.VMEM((CH,), jnp.float32),
        pltpu.VMEM((CH,), jnp.float32),
        pltpu.VMEM((CH,), jnp.float32),
        pltpu.VMEM((CH,), jnp.int32),
        pltpu.VMEM((CH,), jnp.int32),
        pltpu.VMEM((CH,), jnp.float32),
        pltpu.VMEM((CH,), jnp.float32),
        pltpu.VMEM((CH,), jnp.int32),
        pltpu.VMEM((CH,), jnp.int32),
        pltpu.SemaphoreType.DMA,
        pltpu.SemaphoreType.DMA,
    ],
    compiler_params=_sc_params_nl,
)


def _mid_body(p0, p1, A0, A1, ids, embp, W1, b1, W2, oA, oB, dinv):
    deg = 1.0 + p0[...] + p1[...]
    di = lax.rsqrt(deg)
    t1w = jnp.dot(embp[...], W1[...], preferred_element_type=jnp.float32)
    oh = (ids[...] == lax.broadcasted_iota(jnp.int32, (BN, VPAD), 1))
    t1 = jnp.dot(oh.astype(jnp.float32), t1w,
                 preferred_element_type=jnp.float32)
    e64 = jnp.dot(A0[...] + A1[...], t1w, preferred_element_type=jnp.float32)
    h1 = jnp.maximum(di * (e64 + di * t1) + b1[...], 0.0)
    u2 = jnp.dot(h1, W2[...], preferred_element_type=jnp.float32) * di
    oA[...] = u2[:, :HH]
    oB[...] = u2[:, HH:]
    dinv[...] = di


_mid_call = pl.pallas_call(
    _mid_body,
    grid=(GRID,),
    in_specs=[
        pl.BlockSpec((BN, 1), lambda i: (i, 0)),
        pl.BlockSpec((BN, 1), lambda i: (i, 0)),
        pl.BlockSpec((BN, HH), lambda i: (i, 0)),
        pl.BlockSpec((BN, HH), lambda i: (i, 0)),
        pl.BlockSpec((BN, 1), lambda i: (i, 0)),
        pl.BlockSpec((VPAD, H), lambda i: (0, 0)),
        pl.BlockSpec((H, H), lambda i: (0, 0)),
        pl.BlockSpec((1, H), lambda i: (0, 0)),
        pl.BlockSpec((H, H), lambda i: (0, 0)),
    ],
    out_specs=[
        pl.BlockSpec((BN, HH), lambda i: (i, 0)),
        pl.BlockSpec((BN, HH), lambda i: (i, 0)),
        pl.BlockSpec((BN, 1), lambda i: (i, 0)),
    ],
    out_shape=[
        jax.ShapeDtypeStruct((N, HH), jnp.float32),
        jax.ShapeDtypeStruct((N, HH), jnp.float32),
        jax.ShapeDtypeStruct((N, 1), jnp.float32),
    ],
)


def _tail_body(aA, aB, uA, uB, dinv, b2, batch, W3, b3, w4r, b4,
               out, sums, cnts):
    i = pl.program_id(0)
    di = dinv[...]
    acc = jnp.concatenate([aA[...], aB[...]], axis=1)
    u = jnp.concatenate([uA[...], uB[...]], axis=1)
    h2 = jnp.maximum(di * (acc + u) + b2[...], 0.0)
    oh = (batch[...] == lax.broadcasted_iota(jnp.int32, (BN, G), 1))
    ohf = oh.astype(jnp.float32)
    ps = lax.dot_general(ohf, h2, (((0,), (0,)), ((), ())),
                         preferred_element_type=jnp.float32)
    pc = lax.dot_general(ohf, jnp.ones((BN, 1), jnp.float32),
                         (((0,), (0,)), ((), ())),
                         preferred_element_type=jnp.float32)

    @pl.when(i == 0)
    def _():
        sums[...] = ps
        cnts[...] = pc

    @pl.when(i > 0)
    def _():
        sums[...] += ps
        cnts[...] += pc

    @pl.when(i == GRID - 1)
    def _():
        pooled = sums[...] / jnp.maximum(cnts[...], 1.0)
        hid = jnp.maximum(
            jnp.dot(pooled, W3[...], preferred_element_type=jnp.float32)
            + b3[...], 0.0)
        out[...] = jnp.sum(hid * w4r[...], axis=1, keepdims=True) + b4[...]


_tail_call = pl.pallas_call(
    _tail_body,
    grid=(GRID,),
    in_specs=[
        pl.BlockSpec((BN, HH), lambda i: (i, 0)),
        pl.BlockSpec((BN, HH), lambda i: (i, 0)),
        pl.BlockSpec((BN, HH), lambda i: (i, 0)),
        pl.BlockSpec((BN, HH), lambda i: (i, 0)),
        pl.BlockSpec((BN, 1), lambda i: (i, 0)),
        pl.BlockSpec((1, H), lambda i: (0, 0)),
        pl.BlockSpec((BN, 1), lambda i: (i, 0)),
        pl.BlockSpec((H, H), lambda i: (0, 0)),
        pl.BlockSpec((1, H), lambda i: (0, 0)),
        pl.BlockSpec((1, H), lambda i: (0, 0)),
        pl.BlockSpec((1, 1), lambda i: (0, 0)),
    ],
    out_specs=[pl.BlockSpec((G, 1), lambda i: (0, 0))],
    out_shape=[jax.ShapeDtypeStruct((G, 1), jnp.float32)],
    scratch_shapes=[
        pltpu.VMEM((G, G), jnp.float32),
        pltpu.VMEM((G, 1), jnp.float32),
    ],
)


def kernel(x, edge_index, batch, emb, W1, b1, W2, b2, W3, b3, W4, b4):
    src = edge_index[0]
    dst = edge_index[1]
    srcp = jnp.concatenate(
        [src, jnp.zeros((E_PAD - E,), jnp.int32)]).reshape(E_PAD // CH, CH)
    dstp = jnp.concatenate(
        [dst, jnp.full((E_PAD - E,), N, jnp.int32)]).reshape(E_PAD // CH, CH)

    zeros1 = jnp.zeros((RPS,), jnp.float32)
    ones1 = jnp.ones((CH,), jnp.float32)
    zrows = jnp.zeros((WB, HH), jnp.float32)
    zf = jnp.zeros((RPS * HH,), jnp.float32)

    p0f, p1f = _deg_call(dstp, zeros1, ones1)

    A0f, A1f = _conv1_call(srcp, dstp, p0f, p1f, x.reshape(N), zf)

    embp = jnp.zeros((VPAD, H), jnp.float32).at[:emb.shape[0]].set(emb)
    u2A, u2B, dinv = _mid_call(
        p0f[:N].reshape(N, 1), p1f[:N].reshape(N, 1),
        A0f.reshape(ACC_N, HH)[:N], A1f.reshape(ACC_N, HH)[:N],
        x, embp, W1, b1.reshape(1, H), W2)

    a2A, a2B = _conv_call(u2A, u2B, srcp, dstp, zrows)
    (out,) = _tail_call(a2A[:N], a2B[:N], u2A, u2B, dinv,
                        b2.reshape(1, H), batch.reshape(N, 1),
                        W3, b3.reshape(1, H), W4.reshape(1, H),
                        b4.reshape(1, 1))
    return out
